# baseline (speedup 1.0000x reference)
import dataclasses
import functools

import jax
import jax.numpy as jnp
from jax import lax
from jax.experimental import pallas as pl
from jax.experimental.pallas import tpu as pltpu
from jax.experimental.pallas import tpu_sc as plsc

NC, NS, L = 2, 16, 16
NW = NC * NS


@functools.lru_cache(maxsize=None)
def _build(E, H, R, W):
    EW = E // NW
    C = EW // W
    HW = H // 2
    mesh = plsc.VectorSubcoreMesh(
        core_axis_name="c", subcore_axis_name="s", num_cores=NC, num_subcores=NS
    )
    cp = pltpu.CompilerParams()
    if "needs_layout_passes" in pltpu.CompilerParams.__dataclass_fields__:
        cp = dataclasses.replace(cp, needs_layout_passes=False)

    @functools.partial(
        pl.kernel,
        compiler_params=cp,
        out_type=jax.ShapeDtypeStruct((NW, C, W), jnp.float32),
        mesh=mesh,
        scratch_types=[
            pltpu.VMEM((C, W), jnp.int32),
            pltpu.VMEM((C, W), jnp.int32),
            pltpu.VMEM((C, W), jnp.int32),
            pltpu.VMEM((W, H), jnp.int32),
            pltpu.VMEM((W, H), jnp.int32),
            pltpu.VMEM((W, H), jnp.int32),
            pltpu.VMEM((W, H), jnp.int32),
            pltpu.VMEM((W, H), jnp.int32),
            pltpu.VMEM((W, H), jnp.int32),
            pltpu.VMEM((R * HW,), jnp.int32),
            pltpu.VMEM((C, W), jnp.float32),
            pltpu.SemaphoreType.DMA,
            pltpu.SemaphoreType.DMA,
            pltpu.SemaphoreType.DMA,
        ],
    )
    def k(zp_hbm, src_hbm, dst_hbm, typ_hbm, rel_hbm, out_hbm,
          src_v, dst_v, typ_v,
          srowsA, drowsA, srowsB, drowsB, srowsC, drowsC,
          rel_v, out_v, semA, semB, semC):
        wid = lax.axis_index("s") * NC + lax.axis_index("c")
        pltpu.sync_copy(src_hbm.at[wid], src_v)
        pltpu.sync_copy(dst_hbm.at[wid], dst_v)
        pltpu.sync_copy(typ_hbm.at[wid], typ_v)
        pltpu.sync_copy(rel_hbm, rel_v)
        lanes = lax.iota(jnp.int32, L)
        m_last = lanes == (L - 1)

        def start(kk, srows, drows, sem):
            pltpu.async_copy(zp_hbm.at[src_v.at[kk]], srows, sem)
            pltpu.async_copy(zp_hbm.at[dst_v.at[kk]], drows, sem)

        def drain(srows, drows, sem):
            pltpu.make_async_copy(zp_hbm.at[src_v.at[0]], srows, sem).wait()
            pltpu.make_async_copy(zp_hbm.at[dst_v.at[0]], drows, sem).wait()

        def compute(kk, srows, drows):
            @pl.loop(0, W // L)
            def _group(g):
                sl = pl.ds(g * L, L)
                tv = typ_v[kk, sl]
                colb = jnp.broadcast_to(g * L, (L,))
                fmt = plsc.PackFormat.INTERLEAVED
                for u in range(L):
                    e = g * L + u
                    t = tv[u]
                    a0 = a1 = None
                    for q in range(HW // L):
                        sab = plsc.bitcast(
                            srows[e, pl.ds(q * L, L)], jnp.bfloat16)
                        dab = plsc.bitcast(
                            drows[e, pl.ds(q * L, L)], jnp.bfloat16)
                        rab = plsc.bitcast(
                            rel_v[pl.ds(t * HW + q * L, L)], jnp.bfloat16)
                        prod = sab * dab * rab
                        p0, p1 = plsc.unpack(prod, format=fmt)
                        a0 = p0 if a0 is None else a0 + p0
                        a1 = p1 if a1 is None else a1 + p1
                    c = plsc.cumsum(a0 + a1)
                    col = colb + u
                    plsc.store_scatter(out_v.at[kk], [col], c, mask=m_last)

        bufs = (
            (srowsA, drowsA, semA),
            (srowsB, drowsB, semB),
            (srowsC, drowsC, semC),
        )
        start(0, srowsA, drowsA, semA)
        start(1, srowsB, drowsB, semB)

        @pl.loop(0, C)
        def _chunk(kk):
            m = kk % 3
            for b in range(3):
                sr, dr, sm = bufs[b]
                s2, d2, m2 = bufs[(b + 2) % 3]

                @pl.when(m == b)
                def _(sr=sr, dr=dr, sm=sm, s2=s2, d2=d2, m2=m2):
                    drain(sr, dr, sm)

                    @pl.when(kk + 2 < C)
                    def _():
                        start(kk + 2, s2, d2, m2)

                    compute(kk, sr, dr)

        pltpu.sync_copy(out_v, out_hbm.at[wid])

    return k


def kernel(z, edge_index, edge_type, rel_emb):
    E = edge_type.shape[0]
    H = z.shape[1]
    R = rel_emb.shape[0]
    W = 80
    C = E // (NW * W)
    src = edge_index[0].astype(jnp.int32).reshape(NW, C, W)
    dst = edge_index[1].astype(jnp.int32).reshape(NW, C, W)
    typ = edge_type.astype(jnp.int32).reshape(NW, C, W)
    n = z.shape[0]
    zw = lax.bitcast_convert_type(
        z.astype(jnp.bfloat16).reshape(n, H // 2, 2), jnp.int32
    )
    z_tab = jnp.concatenate([zw, zw], axis=1)
    rel_i32 = lax.bitcast_convert_type(
        rel_emb.astype(jnp.bfloat16).reshape(R, H // 2, 2), jnp.int32
    ).reshape(R * (H // 2))
    out = _build(E, H, R, W)(z_tab, src, dst, typ, rel_i32)
    return out.reshape(E)


# speedup vs baseline: 1.2016x; 1.2016x over previous
import dataclasses
import functools

import jax
import jax.numpy as jnp
from jax import lax
from jax.experimental import pallas as pl
from jax.experimental.pallas import tpu as pltpu
from jax.experimental.pallas import tpu_sc as plsc

NC, NS, L = 2, 16, 16
NW = NC * NS


@functools.lru_cache(maxsize=None)
def _build(E, H, R, W):
    EW = E // NW
    C = EW // W
    HW = H // 2
    mesh = plsc.VectorSubcoreMesh(
        core_axis_name="c", subcore_axis_name="s", num_cores=NC, num_subcores=NS
    )
    cp = pltpu.CompilerParams()
    if "needs_layout_passes" in pltpu.CompilerParams.__dataclass_fields__:
        cp = dataclasses.replace(cp, needs_layout_passes=False)

    @functools.partial(
        pl.kernel,
        compiler_params=cp,
        out_type=jax.ShapeDtypeStruct((NW, 1, EW), jnp.float32),
        mesh=mesh,
        scratch_types=[
            pltpu.VMEM((C, W), jnp.int32),
            pltpu.VMEM((C, W), jnp.int32),
            pltpu.VMEM((C, W), jnp.int32),
            pltpu.VMEM((W, H), jnp.int32),
            pltpu.VMEM((W, H), jnp.int32),
            pltpu.VMEM((W, H), jnp.int32),
            pltpu.VMEM((W, H), jnp.int32),
            pltpu.VMEM((W, H), jnp.int32),
            pltpu.VMEM((W, H), jnp.int32),
            pltpu.VMEM((R * HW,), jnp.int32),
            pltpu.VMEM((1, EW), jnp.float32),
            pltpu.VMEM((L * (L + 1),), jnp.float32),
            pltpu.SemaphoreType.DMA,
            pltpu.SemaphoreType.DMA,
            pltpu.SemaphoreType.DMA,
        ],
    )
    def k(zp_hbm, src_hbm, dst_hbm, typ_hbm, rel_hbm, out_hbm,
          src_v, dst_v, typ_v,
          srowsA, drowsA, srowsB, drowsB, srowsC, drowsC,
          rel_v, out_v, part_v, semA, semB, semC):
        wid = lax.axis_index("s") * NC + lax.axis_index("c")
        pltpu.sync_copy(src_hbm.at[wid], src_v)
        pltpu.sync_copy(dst_hbm.at[wid], dst_v)
        pltpu.sync_copy(typ_hbm.at[wid], typ_v)
        pltpu.sync_copy(rel_hbm, rel_v)
        lanes = lax.iota(jnp.int32, L)
        m_last = lanes == (L - 1)

        def start(kk, srows, drows, sem):
            pltpu.async_copy(zp_hbm.at[src_v.at[kk]], srows, sem)
            pltpu.async_copy(zp_hbm.at[dst_v.at[kk]], drows, sem)

        def drain(srows, drows, sem):
            pltpu.make_async_copy(zp_hbm.at[src_v.at[0]], srows, sem).wait()
            pltpu.make_async_copy(zp_hbm.at[dst_v.at[0]], drows, sem).wait()

        tr_idx = lanes * (L + 1)

        def compute(kk, srows, drows):
            @pl.loop(0, W // L)
            def _group(g):
                sl = pl.ds(g * L, L)
                tv = typ_v[kk, sl]
                fmt = plsc.PackFormat.INTERLEAVED
                for u in range(L):
                    e = g * L + u
                    t = tv[u]
                    a0 = a1 = None
                    for q in range(HW // L):
                        sab = plsc.bitcast(
                            srows[e, pl.ds(q * L, L)], jnp.bfloat16)
                        dab = plsc.bitcast(
                            drows[e, pl.ds(q * L, L)], jnp.bfloat16)
                        rab = plsc.bitcast(
                            rel_v[pl.ds(t * HW + q * L, L)], jnp.bfloat16)
                        prod = sab * dab * rab
                        p0, p1 = plsc.unpack(prod, format=fmt)
                        a0 = p0 if a0 is None else a0 + p0
                        a1 = p1 if a1 is None else a1 + p1
                    part_v[pl.ds(u * (L + 1), L)] = a0 + a1
                b0 = b1 = None
                for l in range(L):
                    v = plsc.load_gather(part_v, [tr_idx + l])
                    if l % 2 == 0:
                        b0 = v if b0 is None else b0 + v
                    else:
                        b1 = v if b1 is None else b1 + v
                out_v[0, pl.ds(kk * W + g * L, L)] = b0 + b1

        bufs = (
            (srowsA, drowsA, semA),
            (srowsB, drowsB, semB),
            (srowsC, drowsC, semC),
        )
        start(0, srowsA, drowsA, semA)
        start(1, srowsB, drowsB, semB)

        @pl.loop(0, C)
        def _chunk(kk):
            m = kk % 3
            for b in range(3):
                sr, dr, sm = bufs[b]
                s2, d2, m2 = bufs[(b + 2) % 3]

                @pl.when(m == b)
                def _(sr=sr, dr=dr, sm=sm, s2=s2, d2=d2, m2=m2):
                    drain(sr, dr, sm)

                    @pl.when(kk + 2 < C)
                    def _():
                        start(kk + 2, s2, d2, m2)

                    compute(kk, sr, dr)

        pltpu.sync_copy(out_v, out_hbm.at[wid])

    return k


def kernel(z, edge_index, edge_type, rel_emb):
    E = edge_type.shape[0]
    H = z.shape[1]
    R = rel_emb.shape[0]
    W = 80
    C = E // (NW * W)
    src = edge_index[0].astype(jnp.int32).reshape(NW, C, W)
    dst = edge_index[1].astype(jnp.int32).reshape(NW, C, W)
    typ = edge_type.astype(jnp.int32).reshape(NW, C, W)
    n = z.shape[0]
    zw = lax.bitcast_convert_type(
        z.astype(jnp.bfloat16).reshape(n, H // 2, 2), jnp.int32
    )
    z_tab = jnp.concatenate([zw, zw], axis=1)
    rel_i32 = lax.bitcast_convert_type(
        rel_emb.astype(jnp.bfloat16).reshape(R, H // 2, 2), jnp.int32
    ).reshape(R * (H // 2))
    out = _build(E, H, R, W)(z_tab, src, dst, typ, rel_i32)
    return out.reshape(E)


# speedup vs baseline: 1.4720x; 1.2251x over previous
import dataclasses
import functools

import jax
import jax.numpy as jnp
from jax import lax
from jax.experimental import pallas as pl
from jax.experimental.pallas import tpu as pltpu
from jax.experimental.pallas import tpu_sc as plsc

NC, NS, L = 2, 16, 16
NW = NC * NS


@functools.lru_cache(maxsize=None)
def _build(E, H, R, W):
    EW = E // NW
    C = EW // W
    HW = H // 2
    mesh = plsc.VectorSubcoreMesh(
        core_axis_name="c", subcore_axis_name="s", num_cores=NC, num_subcores=NS
    )
    cp = pltpu.CompilerParams()
    if "needs_layout_passes" in pltpu.CompilerParams.__dataclass_fields__:
        cp = dataclasses.replace(cp, needs_layout_passes=False)

    @functools.partial(
        pl.kernel,
        compiler_params=cp,
        out_type=jax.ShapeDtypeStruct((NW, 1, EW), jnp.float32),
        mesh=mesh,
        scratch_types=[
            pltpu.VMEM((C, W), jnp.int32),
            pltpu.VMEM((C, W), jnp.int32),
            pltpu.VMEM((C, W), jnp.int32),
            pltpu.VMEM((W, H), jnp.int32),
            pltpu.VMEM((W, H), jnp.int32),
            pltpu.VMEM((W, H), jnp.int32),
            pltpu.VMEM((W, H), jnp.int32),
            pltpu.VMEM((W, H), jnp.int32),
            pltpu.VMEM((W, H), jnp.int32),
            pltpu.VMEM((R * HW,), jnp.int32),
            pltpu.VMEM((1, EW), jnp.float32),
            pltpu.VMEM((L * (L + 1),), jnp.float32),
            pltpu.SemaphoreType.DMA,
            pltpu.SemaphoreType.DMA,
            pltpu.SemaphoreType.DMA,
        ],
    )
    def k(zp_hbm, src_hbm, dst_hbm, typ_hbm, rel_hbm, out_hbm,
          src_v, dst_v, typ_v,
          srowsA, drowsA, srowsB, drowsB, srowsC, drowsC,
          rel_v, out_v, part_v, semA, semB, semC):
        wid = lax.axis_index("s") * NC + lax.axis_index("c")
        pltpu.sync_copy(src_hbm.at[wid], src_v)
        pltpu.sync_copy(dst_hbm.at[wid], dst_v)
        pltpu.sync_copy(typ_hbm.at[wid], typ_v)
        pltpu.sync_copy(rel_hbm, rel_v)
        lanes = lax.iota(jnp.int32, L)
        m_last = lanes == (L - 1)

        def start(kk, srows, drows, sem):
            pltpu.async_copy(zp_hbm.at[src_v.at[kk]], srows, sem)
            pltpu.async_copy(zp_hbm.at[dst_v.at[kk]], drows, sem)

        def drain(srows, drows, sem):
            pltpu.make_async_copy(zp_hbm.at[src_v.at[0]], srows, sem).wait()
            pltpu.make_async_copy(zp_hbm.at[dst_v.at[0]], drows, sem).wait()

        tr_idx = lanes * (L + 1)

        def compute(kk, srows, drows):
            @pl.loop(0, W // L)
            def _group(g):
                sl = pl.ds(g * L, L)
                tv = typ_v[kk, sl]
                fmt = plsc.PackFormat.INTERLEAVED

                def loads(u):
                    e = g * L + u
                    t = tv[u]
                    out = []
                    for q in range(HW // L):
                        out.append((
                            plsc.bitcast(
                                srows[e, pl.ds(q * L, L)], jnp.bfloat16),
                            plsc.bitcast(
                                drows[e, pl.ds(q * L, L)], jnp.bfloat16),
                            plsc.bitcast(
                                rel_v[pl.ds(t * HW + q * L, L)], jnp.bfloat16),
                        ))
                    return out

                pending = loads(0)
                for u in range(L):
                    nxt = loads(u + 1) if u + 1 < L else None
                    a0 = a1 = None
                    for sab, dab, rab in pending:
                        prod = sab * dab * rab
                        p0, p1 = plsc.unpack(prod, format=fmt)
                        a0 = p0 if a0 is None else a0 + p0
                        a1 = p1 if a1 is None else a1 + p1
                    part_v[pl.ds(u * (L + 1), L)] = a0 + a1
                    pending = nxt
                b0 = b1 = None
                for l in range(L):
                    v = plsc.load_gather(part_v, [tr_idx + l])
                    if l % 2 == 0:
                        b0 = v if b0 is None else b0 + v
                    else:
                        b1 = v if b1 is None else b1 + v
                out_v[0, pl.ds(kk * W + g * L, L)] = b0 + b1

        bufs = (
            (srowsA, drowsA, semA),
            (srowsB, drowsB, semB),
            (srowsC, drowsC, semC),
        )
        start(0, srowsA, drowsA, semA)
        start(1, srowsB, drowsB, semB)

        @pl.loop(0, C)
        def _chunk(kk):
            m = kk % 3
            for b in range(3):
                sr, dr, sm = bufs[b]
                s2, d2, m2 = bufs[(b + 2) % 3]

                @pl.when(m == b)
                def _(sr=sr, dr=dr, sm=sm, s2=s2, d2=d2, m2=m2):
                    drain(sr, dr, sm)

                    @pl.when(kk + 2 < C)
                    def _():
                        start(kk + 2, s2, d2, m2)

                    compute(kk, sr, dr)

        pltpu.sync_copy(out_v, out_hbm.at[wid])

    return k


def kernel(z, edge_index, edge_type, rel_emb):
    E = edge_type.shape[0]
    H = z.shape[1]
    R = rel_emb.shape[0]
    W = 80
    C = E // (NW * W)
    src = edge_index[0].astype(jnp.int32).reshape(NW, C, W)
    dst = edge_index[1].astype(jnp.int32).reshape(NW, C, W)
    typ = edge_type.astype(jnp.int32).reshape(NW, C, W)
    n = z.shape[0]
    zw = lax.bitcast_convert_type(
        z.astype(jnp.bfloat16).reshape(n, H // 2, 2), jnp.int32
    )
    z_tab = jnp.concatenate([zw, zw], axis=1)
    rel_i32 = lax.bitcast_convert_type(
        rel_emb.astype(jnp.bfloat16).reshape(R, H // 2, 2), jnp.int32
    ).reshape(R * (H // 2))
    out = _build(E, H, R, W)(z_tab, src, dst, typ, rel_i32)
    return out.reshape(E)
